# baseline (speedup 1.0000x reference)
import functools

import jax
import jax.numpy as jnp
from jax.experimental import pallas as pl
from jax.experimental.pallas import tpu as pltpu


def _round_up(x: int, m: int) -> int:
    return ((x + m - 1) // m) * m


def _pick_tile_e(num_edges: int) -> int:
    for cand in (32000, 16000, 8192, 6400, 4096, 3200, 2048, 1600, 1280, 1024,
                 640, 512, 256, 128):
        if num_edges % cand == 0:
            return cand
    return 2048


def _edge_interp_kernel(grid_ref, rij_ref, et_ref, tab_ref, gidx_ref,
                        h_ref, s_ref, *, num_bonds, n_sub, m, s_row0):
    xmin = grid_ref[0]
    inv_dx = grid_ref[1]

    r = rij_ref[...]
    et = et_ref[...]
    u = (r - xmin) * inv_dx

    d = (u - gidx_ref[...]).astype(jnp.bfloat16)
    one = jnp.bfloat16(1.0)
    w = jnp.maximum(one - jnp.abs(d), jnp.bfloat16(0.0))

    slab = jnp.dot(tab_ref[...], w,
                   preferred_element_type=jnp.float32)

    slabs = [slab[b * n_sub:(b + 1) * n_sub, :] for b in range(num_bonds)]
    if num_bonds & (num_bonds - 1) == 0:
        bit = 1
        while len(slabs) > 1:
            slabs = [jnp.where((et & bit) == 0, slabs[i], slabs[i + 1])
                     for i in range(0, len(slabs), 2)]
            bit <<= 1
        sel = slabs[0]
    else:
        sel = jnp.where(et == 0, slabs[0], 0.0)
        for b in range(1, num_bonds):
            sel = jnp.where(et == b, slabs[b], sel)

    h_ref[...] = sel[0:m, :].astype(h_ref.dtype)
    s_ref[...] = sel[s_row0:s_row0 + m, :].astype(s_ref.dtype)


def _interp_edges(rij, edge_type, xx, hopping, overlap):
    e = rij.shape[0]
    b, m, g = hopping.shape

    n_sub = 2 * _round_up(m, 16)
    s_row0 = n_sub // 2
    g_pad = _round_up(g, 8)

    tab = jnp.zeros((b, n_sub, g_pad), jnp.float32)
    tab = tab.at[:, :m, :g].set(hopping.astype(jnp.float32))
    tab = tab.at[:, s_row0:s_row0 + m, :g].set(overlap.astype(jnp.float32))
    tab = tab.reshape(b * n_sub, g_pad).astype(jnp.bfloat16)

    tile_e = _pick_tile_e(e)
    gidx = jnp.broadcast_to(
        jnp.arange(g_pad, dtype=jnp.float32)[:, None], (g_pad, tile_e))
    e_pad = _round_up(max(e, 1), tile_e)
    rij_p = rij.astype(jnp.float32).reshape(1, e)
    et_p = edge_type.astype(jnp.int32).reshape(1, e)
    if e_pad != e:
        rij_p = jnp.pad(rij_p, ((0, 0), (0, e_pad - e)))
        et_p = jnp.pad(et_p, ((0, 0), (0, e_pad - e)))

    xx = xx.astype(jnp.float32)
    inv_dx = jnp.float32(g - 1) / (xx[-1] - xx[0])
    grid_info = jnp.stack([xx[0], inv_dx])

    h, s = pl.pallas_call(
        functools.partial(_edge_interp_kernel, num_bonds=b, n_sub=n_sub, m=m,
                          s_row0=s_row0),
        out_shape=(jax.ShapeDtypeStruct((m, e_pad), jnp.bfloat16),
                   jax.ShapeDtypeStruct((m, e_pad), jnp.bfloat16)),
        grid=(e_pad // tile_e,),
        in_specs=[
            pl.BlockSpec(memory_space=pltpu.MemorySpace.SMEM),
            pl.BlockSpec((1, tile_e), lambda i: (0, i)),
            pl.BlockSpec((1, tile_e), lambda i: (0, i)),
            pl.BlockSpec((b * n_sub, g_pad), lambda i: (0, 0)),
            pl.BlockSpec((g_pad, tile_e), lambda i: (0, 0)),
        ],
        out_specs=(pl.BlockSpec((m, tile_e), lambda i: (0, i)),
                   pl.BlockSpec((m, tile_e), lambda i: (0, i))),
        compiler_params=pltpu.CompilerParams(
            dimension_semantics=("parallel",)),
    )(grid_info, rij_p, et_p, tab, gidx)

    return (h[:, :e].T.astype(jnp.float32), s[:, :e].T.astype(jnp.float32))


def kernel(distance_param, hopping_param, overlap_param, onsite_param,
           edge_length, edge_type, atom_type, pbc):
    h, s = _interp_edges(edge_length, edge_type, distance_param,
                         hopping_param, overlap_param)

    out = {
        "edge_length": edge_length,
        "edge_type": edge_type,
        "atom_type": atom_type,
        "pbc": pbc,
    }
    out["edge_features"] = h
    out["edge_overlap"] = s
    n_onsite = onsite_param.shape[1]
    out["node_overlap"] = jnp.ones((atom_type.shape[0], n_onsite), jnp.float32)

    onsite_table = onsite_param[..., 0]
    out["node_features"] = jnp.take(onsite_table, atom_type.astype(jnp.int32),
                                    axis=0)
    out["node_soc_switch"] = jnp.zeros((pbc.shape[0], 1), dtype=jnp.bool_)
    return out


# speedup vs baseline: 1.3266x; 1.3266x over previous
import functools

import jax
import jax.numpy as jnp
from jax.experimental import pallas as pl
from jax.experimental.pallas import tpu as pltpu


def _round_up(x: int, m: int) -> int:
    return ((x + m - 1) // m) * m


def _pick_tile_e(num_edges: int) -> int:
    for cand in (32000, 16000, 8192, 6400, 4096, 3200, 2048, 1600, 1280, 1024,
                 640, 512, 256, 128):
        if num_edges % cand == 0:
            return cand
    return 2048


def _edge_interp_kernel(grid_ref, rij_ref, et_ref, tab_ref, gidx_ref,
                        h_ref, s_ref, *, num_bonds, n_sub, m, s_row0):
    xmin = grid_ref[0]
    inv_dx = grid_ref[1]

    r = rij_ref[...]
    et = et_ref[...]
    u = (r - xmin) * inv_dx

    d = (u - gidx_ref[...]).astype(jnp.bfloat16)
    one = jnp.bfloat16(1.0)
    w = jnp.maximum(one - jnp.abs(d), jnp.bfloat16(0.0))

    slab = jnp.dot(tab_ref[...], w,
                   preferred_element_type=jnp.float32)

    slabs = [slab[b * n_sub:(b + 1) * n_sub, :] for b in range(num_bonds)]
    if num_bonds & (num_bonds - 1) == 0:
        bit = 1
        while len(slabs) > 1:
            slabs = [jnp.where((et & bit) == 0, slabs[i], slabs[i + 1])
                     for i in range(0, len(slabs), 2)]
            bit <<= 1
        sel = slabs[0]
    else:
        sel = jnp.where(et == 0, slabs[0], 0.0)
        for b in range(1, num_bonds):
            sel = jnp.where(et == b, slabs[b], sel)

    h_ref[...] = sel[0:m, :]
    s_ref[...] = sel[s_row0:s_row0 + m, :]


def _interp_edges(rij, edge_type, xx, hopping, overlap):
    e = rij.shape[0]
    b, m, g = hopping.shape

    n_sub = 2 * _round_up(m, 16)
    s_row0 = n_sub // 2
    g_pad = _round_up(g, 8)

    tab = jnp.zeros((b, n_sub, g_pad), jnp.float32)
    tab = tab.at[:, :m, :g].set(hopping.astype(jnp.float32))
    tab = tab.at[:, s_row0:s_row0 + m, :g].set(overlap.astype(jnp.float32))
    tab = tab.reshape(b * n_sub, g_pad).astype(jnp.bfloat16)

    tile_e = _pick_tile_e(e)
    gidx = jnp.broadcast_to(
        jnp.arange(g_pad, dtype=jnp.float32)[:, None], (g_pad, tile_e))
    e_pad = _round_up(max(e, 1), tile_e)
    rij_p = rij.astype(jnp.float32).reshape(1, e)
    et_p = edge_type.astype(jnp.int32).reshape(1, e)
    if e_pad != e:
        rij_p = jnp.pad(rij_p, ((0, 0), (0, e_pad - e)))
        et_p = jnp.pad(et_p, ((0, 0), (0, e_pad - e)))

    xx = xx.astype(jnp.float32)
    inv_dx = jnp.float32(g - 1) / (xx[-1] - xx[0])
    grid_info = jnp.stack([xx[0], inv_dx])

    h, s = pl.pallas_call(
        functools.partial(_edge_interp_kernel, num_bonds=b, n_sub=n_sub, m=m,
                          s_row0=s_row0),
        out_shape=(jax.ShapeDtypeStruct((m, e_pad), jnp.float32),
                   jax.ShapeDtypeStruct((m, e_pad), jnp.float32)),
        grid=(e_pad // tile_e,),
        in_specs=[
            pl.BlockSpec(memory_space=pltpu.MemorySpace.SMEM),
            pl.BlockSpec((1, tile_e), lambda i: (0, i)),
            pl.BlockSpec((1, tile_e), lambda i: (0, i)),
            pl.BlockSpec((b * n_sub, g_pad), lambda i: (0, 0)),
            pl.BlockSpec((g_pad, tile_e), lambda i: (0, 0)),
        ],
        out_specs=(pl.BlockSpec((m, tile_e), lambda i: (0, i)),
                   pl.BlockSpec((m, tile_e), lambda i: (0, i))),
        compiler_params=pltpu.CompilerParams(
            dimension_semantics=("parallel",)),
    )(grid_info, rij_p, et_p, tab, gidx)

    return h[:, :e].T, s[:, :e].T


def kernel(distance_param, hopping_param, overlap_param, onsite_param,
           edge_length, edge_type, atom_type, pbc):
    h, s = _interp_edges(edge_length, edge_type, distance_param,
                         hopping_param, overlap_param)

    out = {
        "edge_length": edge_length,
        "edge_type": edge_type,
        "atom_type": atom_type,
        "pbc": pbc,
    }
    out["edge_features"] = h
    out["edge_overlap"] = s
    n_onsite = onsite_param.shape[1]
    out["node_overlap"] = jnp.ones((atom_type.shape[0], n_onsite), jnp.float32)

    onsite_table = onsite_param[..., 0]
    out["node_features"] = jnp.take(onsite_table, atom_type.astype(jnp.int32),
                                    axis=0)
    out["node_soc_switch"] = jnp.zeros((pbc.shape[0], 1), dtype=jnp.bool_)
    return out


# speedup vs baseline: 2.0555x; 1.5495x over previous
import functools

import jax
import jax.numpy as jnp
from jax.experimental import pallas as pl
from jax.experimental.pallas import tpu as pltpu


def _round_up(x: int, m: int) -> int:
    return ((x + m - 1) // m) * m


def _pick_tile_e(num_edges: int) -> int:
    for cand in (32000, 16000, 8192, 6400, 4096, 3200, 2048, 1600, 1280, 1024,
                 640, 512, 256, 128):
        if num_edges % cand == 0:
            return cand
    return 2048


def _edge_interp_kernel(grid_ref, rij_ref, et_ref, tab_ref, gidx_ref,
                        h_ref, s_ref, *, num_bonds, n_sub, m, s_row0):
    xmin = grid_ref[0]
    inv_dx = grid_ref[1]

    r = rij_ref[...]
    et = et_ref[...]
    u = (r - xmin) * inv_dx

    k = jnp.floor(u)
    t = (u - k).astype(jnp.bfloat16)
    kb = k.astype(jnp.bfloat16)
    d = (kb - gidx_ref[...]) + t
    one = jnp.bfloat16(1.0)
    w = jnp.maximum(one - jnp.abs(d), jnp.bfloat16(0.0))

    slab = jnp.dot(tab_ref[...], w,
                   preferred_element_type=jnp.float32)

    slabs = [slab[b * n_sub:(b + 1) * n_sub, :] for b in range(num_bonds)]
    if num_bonds & (num_bonds - 1) == 0:
        bit = 1
        while len(slabs) > 1:
            slabs = [jnp.where((et & bit) == 0, slabs[i], slabs[i + 1])
                     for i in range(0, len(slabs), 2)]
            bit <<= 1
        sel = slabs[0]
    else:
        sel = jnp.where(et == 0, slabs[0], 0.0)
        for b in range(1, num_bonds):
            sel = jnp.where(et == b, slabs[b], sel)

    h_ref[...] = sel[0:m, :]
    s_ref[...] = sel[s_row0:s_row0 + m, :]


def _interp_edges(rij, edge_type, xx, hopping, overlap):
    e = rij.shape[0]
    b, m, g = hopping.shape

    n_sub = 2 * _round_up(m, 16)
    s_row0 = n_sub // 2
    g_pad = _round_up(g, 8)

    tab = jnp.zeros((b, n_sub, g_pad), jnp.float32)
    tab = tab.at[:, :m, :g].set(hopping.astype(jnp.float32))
    tab = tab.at[:, s_row0:s_row0 + m, :g].set(overlap.astype(jnp.float32))
    tab = tab.reshape(b * n_sub, g_pad).astype(jnp.bfloat16)

    tile_e = _pick_tile_e(e)
    gidx = jnp.broadcast_to(
        jnp.arange(g_pad, dtype=jnp.bfloat16)[:, None], (g_pad, tile_e))
    e_pad = _round_up(max(e, 1), tile_e)
    rij_p = rij.astype(jnp.float32).reshape(1, e)
    et_p = edge_type.astype(jnp.int32).reshape(1, e)
    if e_pad != e:
        rij_p = jnp.pad(rij_p, ((0, 0), (0, e_pad - e)))
        et_p = jnp.pad(et_p, ((0, 0), (0, e_pad - e)))

    xx = xx.astype(jnp.float32)
    inv_dx = jnp.float32(g - 1) / (xx[-1] - xx[0])
    grid_info = jnp.stack([xx[0], inv_dx])

    h, s = pl.pallas_call(
        functools.partial(_edge_interp_kernel, num_bonds=b, n_sub=n_sub, m=m,
                          s_row0=s_row0),
        out_shape=(jax.ShapeDtypeStruct((m, e_pad), jnp.float32),
                   jax.ShapeDtypeStruct((m, e_pad), jnp.float32)),
        grid=(e_pad // tile_e,),
        in_specs=[
            pl.BlockSpec(memory_space=pltpu.MemorySpace.SMEM),
            pl.BlockSpec((1, tile_e), lambda i: (0, i)),
            pl.BlockSpec((1, tile_e), lambda i: (0, i)),
            pl.BlockSpec((b * n_sub, g_pad), lambda i: (0, 0)),
            pl.BlockSpec((g_pad, tile_e), lambda i: (0, 0)),
        ],
        out_specs=(pl.BlockSpec((m, tile_e), lambda i: (0, i)),
                   pl.BlockSpec((m, tile_e), lambda i: (0, i))),
        compiler_params=pltpu.CompilerParams(
            dimension_semantics=("parallel",)),
    )(grid_info, rij_p, et_p, tab, gidx)

    return h[:, :e].T, s[:, :e].T


def kernel(distance_param, hopping_param, overlap_param, onsite_param,
           edge_length, edge_type, atom_type, pbc):
    h, s = _interp_edges(edge_length, edge_type, distance_param,
                         hopping_param, overlap_param)

    out = {
        "edge_length": edge_length,
        "edge_type": edge_type,
        "atom_type": atom_type,
        "pbc": pbc,
    }
    out["edge_features"] = h
    out["edge_overlap"] = s
    n_onsite = onsite_param.shape[1]
    out["node_overlap"] = jnp.ones((atom_type.shape[0], n_onsite), jnp.float32)

    onsite_table = onsite_param[..., 0]
    at = atom_type.astype(jnp.int32)[:, None]
    n_types = onsite_table.shape[0]
    nf = jnp.broadcast_to(onsite_table[0], (at.shape[0], n_onsite))
    for tt in range(1, n_types):
        nf = jnp.where(at == tt, onsite_table[tt], nf)
    out["node_features"] = nf
    out["node_soc_switch"] = jnp.zeros((pbc.shape[0], 1), dtype=jnp.bool_)
    return out
